# Initial kernel scaffold; baseline (speedup 1.0000x reference)
import jax
import jax.numpy as jnp
from jax.experimental import pallas as pl


def kernel(x, embedding_weight, pos_embedding_weight):
    raise NotImplementedError("write your pallas kernel here")


# speedup vs baseline: 2.8848x; 2.8848x over previous
import functools

import jax
import jax.numpy as jnp
from jax import lax
from jax.experimental import pallas as pl
from jax.experimental.pallas import tpu as pltpu
from jax.experimental.pallas import tpu_sc as plsc

_VOC = 100000
_SEQ = 200
_D = 64
_BATCH = 4096
_TOT = _BATCH * _SEQ

_NC = 2
_NS = 16
_NW = _NC * _NS
_ROWS_PER_W = _TOT // _NW
_C = 128
_NCHUNK = _ROWS_PER_W // _C
_NBUF = 4
_LOOKAHEAD = 2


def _make_sc_call():
    mesh = plsc.VectorSubcoreMesh(
        core_axis_name="c", subcore_axis_name="s",
        num_cores=_NC, num_subcores=_NS)

    @functools.partial(
        pl.kernel,
        out_type=jax.ShapeDtypeStruct((_TOT, _D), jnp.float32),
        mesh=mesh,
        compiler_params=pltpu.CompilerParams(use_tc_tiling_on_sc=False),
        scratch_types=[
            pltpu.VMEM((_NCHUNK, _C), jnp.int32),
            pltpu.VMEM((_SEQ + _C, _D), jnp.float32),
            [pltpu.VMEM((_C, _D), jnp.float32) for _ in range(_NBUF)],
            [pltpu.SemaphoreType.DMA for _ in range(_NBUF)],
            [pltpu.SemaphoreType.DMA for _ in range(_NBUF)],
        ],
    )
    def emb_kernel(idx_hbm, table_hbm, pos_hbm, out_hbm,
                   idx_v, pos_v, bufs, gsems, osems):
        wid = lax.axis_index("s") * _NC + lax.axis_index("c")
        base = wid * _ROWS_PER_W

        pltpu.sync_copy(idx_hbm.at[wid], idx_v)
        pltpu.sync_copy(pos_hbm, pos_v)

        def start_gather(j, b):
            pltpu.async_copy(table_hbm.at[idx_v.at[j]], bufs[b], gsems[b])

        def wait_gather(b):
            pltpu.make_async_copy(
                table_hbm.at[idx_v.at[0]], bufs[b], gsems[b]).wait()

        def start_out(j, b):
            pltpu.async_copy(
                bufs[b], out_hbm.at[pl.ds(base + j * _C, _C)], osems[b])

        def wait_out(b):
            pltpu.make_async_copy(
                bufs[b], out_hbm.at[pl.ds(base, _C)], osems[b]).wait()

        def add_pos(j, b):
            p = lax.rem(j * _C, _SEQ)
            buf = bufs[b]

            def body(r, carry):
                pr = p + r
                for c in range(_D // 16):
                    sl = pl.ds(c * 16, 16)
                    buf[r, sl] = buf[r, sl] + pos_v[pr, sl]
                return carry

            lax.fori_loop(0, _C, body, 0, unroll=2)

        start_gather(0, 0)
        start_gather(1, 1)

        for j in range(4):
            jn = j + _LOOKAHEAD
            bn = jn % _NBUF
            if jn >= _NBUF:
                wait_out(bn)
            start_gather(jn, bn)
            wait_gather(j % _NBUF)
            add_pos(j, j % _NBUF)
            start_out(j, j % _NBUF)

        def outer(t, carry):
            j0 = t * _NBUF
            for k in range(_NBUF):
                j = j0 + k
                jn = j + _LOOKAHEAD
                bn = (k + _LOOKAHEAD) % _NBUF
                wait_out(bn)
                start_gather(jn, bn)
                wait_gather(k)
                add_pos(j, k)
                start_out(j, k)
            return carry

        lax.fori_loop(1, _NCHUNK // _NBUF - 1, outer, 0)

        for j in range(_NCHUNK - 4, _NCHUNK):
            jn = j + _LOOKAHEAD
            b = j % _NBUF
            if jn < _NCHUNK:
                bn = jn % _NBUF
                wait_out(bn)
                start_gather(jn, bn)
            wait_gather(b)
            add_pos(j, b)
            start_out(j, b)

        for b in range(_NBUF):
            wait_out(b)

    return emb_kernel


_emb_call = _make_sc_call()


def kernel(x, embedding_weight, pos_embedding_weight):
    idx = x.astype(jnp.int32).reshape(_NW, _NCHUNK, _C)
    pos_ext = jnp.concatenate(
        [pos_embedding_weight, pos_embedding_weight[:_C]], axis=0)
    out = _emb_call(idx, embedding_weight, pos_ext)
    return out.reshape(_BATCH, _SEQ, _D)
